# baseline (speedup 1.0000x reference)
import functools
import math

import jax
import jax.numpy as jnp
from jax import lax
from jax.experimental import pallas as pl
from jax.experimental import layout as _layout
from jax.experimental.pallas import tpu as pltpu
from jax.experimental.pallas import tpu_sc as plsc

_LANES = 16
_IDX_W = 128


def _embed_sc(tokens_2d, table_pairs):
    n_rows, idx_w = tokens_2d.shape
    n_pairs, two_d = table_pairs.shape
    dim = two_d // 2
    info = plsc.get_sparse_core_info()
    n_workers = info.num_cores * info.num_subcores
    rows_per_w = n_rows // n_workers
    total = n_rows * idx_w

    mesh = plsc.VectorSubcoreMesh(core_axis_name="c", subcore_axis_name="s")

    @functools.partial(
        pl.kernel,
        mesh=mesh,
        out_type=jax.ShapeDtypeStruct((total, dim), jnp.float32),
        scratch_types=[
            pltpu.VMEM((rows_per_w, idx_w), jnp.int32),
            pltpu.VMEM((1, idx_w), jnp.int32),
            pltpu.VMEM((idx_w, two_d), jnp.float32),
            pltpu.VMEM((idx_w, dim), jnp.float32),
            pltpu.SemaphoreType.DMA,
        ],
        compiler_params=pltpu.CompilerParams(use_tc_tiling_on_sc=True),
    )
    def k(tok_hbm, tab_hbm, out_hbm, idx_v, pidx_v, buf_v, obuf_v, sem):
        w = lax.axis_index("s") * info.num_cores + lax.axis_index("c")
        pltpu.sync_copy(tok_hbm.at[pl.ds(w * rows_per_w, rows_per_w)], idx_v)
        tbase = w * rows_per_w * idx_w

        def chunk(j, _):
            for k16 in range(idx_w // _LANES):
                sl = pl.ds(k16 * _LANES, _LANES)
                pidx_v[0, sl] = lax.shift_right_logical(idx_v[j, sl], 1)
            pltpu.async_copy(tab_hbm.at[pidx_v.at[0]], buf_v, sem).wait()

            def grp_body(g, _):
                tv = idx_v[j, pl.ds(g * _LANES, _LANES)]
                for l in range(_LANES):
                    off = (tv[l] & 1) * dim
                    for k16 in range(dim // _LANES):
                        obuf_v[
                            _LANES * g + l, pl.ds(k16 * _LANES, _LANES)
                        ] = buf_v[_LANES * g + l, pl.ds(off + k16 * _LANES, _LANES)]
                return 0

            lax.fori_loop(0, idx_w // _LANES, grp_body, 0)
            pltpu.sync_copy(obuf_v, out_hbm.at[pl.ds(tbase + j * idx_w, idx_w)])
            return 0

        lax.fori_loop(0, rows_per_w, chunk, 0)

    return k(tokens_2d, table_pairs)


def kernel(tokens, embedding_weight):
    b0, b1 = tokens.shape
    vocab, dim = embedding_weight.shape
    scale = math.sqrt(dim)
    toks = tokens.reshape(b0 * b1 // _IDX_W, _IDX_W)
    table_pairs = embedding_weight.reshape(vocab // 2, 2 * dim) * scale
    out = _embed_sc(toks, table_pairs)
    out = out.reshape(b0, b1, dim)
    return _layout.with_layout_constraint(
        out, _layout.Layout(major_to_minor=(0, 1, 2))
    )


# speedup vs baseline: 1.8423x; 1.8423x over previous
import functools
import math

import jax
import jax.numpy as jnp
from jax import lax
from jax.experimental import pallas as pl
from jax.experimental import layout as _layout
from jax.experimental.pallas import tpu as pltpu
from jax.experimental.pallas import tpu_sc as plsc

_LANES = 16
_IDX_W = 128


def _embed_sc(tokens_2d, table_slots, scale):
    n_rows, idx_w = tokens_2d.shape
    vocab, slot_w = table_slots.shape
    dim = slot_w // 2
    info = plsc.get_sparse_core_info()
    n_workers = info.num_cores * info.num_subcores
    rows_per_w = n_rows // n_workers
    total = n_rows * idx_w

    mesh = plsc.VectorSubcoreMesh(core_axis_name="c", subcore_axis_name="s")

    @functools.partial(
        pl.kernel,
        mesh=mesh,
        out_type=jax.ShapeDtypeStruct((total, dim), jnp.float32),
        scratch_types=[
            pltpu.VMEM((rows_per_w, idx_w), jnp.int32),
            pltpu.VMEM((idx_w, slot_w), jnp.float32),
            pltpu.VMEM((idx_w, slot_w), jnp.float32),
            pltpu.VMEM((idx_w, dim), jnp.float32),
            pltpu.SemaphoreType.DMA,
            pltpu.SemaphoreType.DMA,
        ],
        compiler_params=pltpu.CompilerParams(use_tc_tiling_on_sc=True),
    )
    def k(tok_hbm, tab_hbm, out_hbm, idx_v, buf_a, buf_b, obuf_v, sem_a, sem_b):
        def compact_store(buf, j):
            def row_body(r, _):
                for k16 in range(dim // _LANES):
                    sl = pl.ds(k16 * _LANES, _LANES)
                    obuf_v[r, sl] = buf[r, sl] * scale
                return 0

            lax.fori_loop(0, idx_w, row_body, 0)
            pltpu.sync_copy(obuf_v, out_hbm.at[pl.ds(tbase + j * idx_w, idx_w)])

        w = lax.axis_index("s") * info.num_cores + lax.axis_index("c")
        pltpu.sync_copy(tok_hbm.at[pl.ds(w * rows_per_w, rows_per_w)], idx_v)
        tbase = w * rows_per_w * idx_w

        pltpu.async_copy(tab_hbm.at[idx_v.at[0]], buf_a, sem_a)

        def body(m, _):
            j = 2 * m
            h_b = pltpu.async_copy(tab_hbm.at[idx_v.at[j + 1]], buf_b, sem_b)
            pltpu.make_async_copy(tab_hbm.at[idx_v.at[0]], buf_a, sem_a).wait()
            compact_store(buf_a, j)

            @pl.when(m < rows_per_w // 2 - 1)
            def _():
                pltpu.async_copy(tab_hbm.at[idx_v.at[j + 2]], buf_a, sem_a)

            h_b.wait()
            compact_store(buf_b, j + 1)
            return 0

        lax.fori_loop(0, rows_per_w // 2, body, 0)

    return k(tokens_2d, table_slots)


def kernel(tokens, embedding_weight):
    b0, b1 = tokens.shape
    vocab, dim = embedding_weight.shape
    scale = math.sqrt(dim)
    toks = tokens.reshape(b0 * b1 // _IDX_W, _IDX_W)
    table_slots = jnp.concatenate(
        [embedding_weight, jnp.zeros((vocab, dim), jnp.float32)], axis=1
    )
    out = _embed_sc(toks, table_slots, scale)
    out = out.reshape(b0, b1, dim)
    return _layout.with_layout_constraint(
        out, _layout.Layout(major_to_minor=(0, 1, 2))
    )
